# Initial kernel scaffold; baseline (speedup 1.0000x reference)
import jax
import jax.numpy as jnp
from jax.experimental import pallas as pl


def kernel(x):
    raise NotImplementedError("write your pallas kernel here")


# speedup vs baseline: 42.2011x; 42.2011x over previous
import functools

import jax
import jax.numpy as jnp
import numpy as np
from jax import lax
from jax.experimental import pallas as pl
from jax.experimental.pallas import tpu as pltpu
from jax.experimental.pallas import tpu_sc as plsc

H = W = 512
HW = H * W
NSEL = 16
NB = 2048
LO = -0.25
HI = 1.25
WBIN = (HI - LO) / NB
INVW = 1.0 / WBIN
CHUNK = 8192
NCHUNK = HW // CHUNK
ROWS = 128

_K = 8


def _rgb2ycbcr_body(xb, ob):
    r = xb[0, 0]
    g = xb[0, 1]
    b = xb[0, 2]
    y = 0.299 * r + 0.587 * g + 0.114 * b
    ob[0, 0] = y
    ob[0, 1] = (b - y) * 0.564 + 0.5
    ob[0, 2] = (r - y) * 0.713 + 0.5


_rgb2ycbcr = pl.pallas_call(
    _rgb2ycbcr_body,
    grid=(NSEL, H // ROWS),
    in_specs=[pl.BlockSpec((1, 3, ROWS, W), lambda i, j: (i, 0, j, 0))],
    out_specs=pl.BlockSpec((1, 3, ROWS, W), lambda i, j: (i, 0, j, 0)),
    out_shape=jax.ShapeDtypeStruct((NSEL, 3, H, W), jnp.float32),
)


def _recon_body(yb, cbcrb, ob):
    y = yb[0, 0]
    cb = cbcrb[0, 0] - 0.5
    cr = cbcrb[0, 1] - 0.5
    ob[0, 0] = y + 1.403 * cr
    ob[0, 1] = y - 0.714 * cr - 0.344 * cb
    ob[0, 2] = y + 1.773 * cb


_recon = pl.pallas_call(
    _recon_body,
    grid=(NSEL, H // ROWS),
    in_specs=[
        pl.BlockSpec((1, 1, ROWS, W), lambda i, j: (i, 0, j, 0)),
        pl.BlockSpec((1, 2, ROWS, W), lambda i, j: (i, 0, j, 0)),
    ],
    out_specs=pl.BlockSpec((1, 3, ROWS, W), lambda i, j: (i, 0, j, 0)),
    out_shape=jax.ShapeDtypeStruct((NSEL, 3, H, W), jnp.float32),
)


def _sc_body(ycbcr, cbcr_out, hists, buf, hist_lanes, hist_own, hist_ref,
             cum_img, cum_ref, lut):
    c = lax.axis_index("c")
    s = lax.axis_index("s")
    k_img = (s // 2) * 2 + c
    chan = s % 2
    k_ref = (k_img + 8) % 16
    row = k_img * 2 + chan
    prow = k_ref * 2 + chan

    lane = lax.iota(jnp.int32, 16)
    lane_off = lane * NB
    ones = jnp.full((16,), 1.0, jnp.float32)
    zeros = jnp.zeros((16,), jnp.float32)
    in_base = (k_img * 3 + 1 + chan) * HW
    out_base = (k_img * 2 + chan) * HW

    def zero_body(i, _):
        hist_lanes[pl.ds(i * 16, 16)] = zeros
        return _

    lax.fori_loop(0, NB, zero_body, None)

    def hist_chunk(ci, _):
        pltpu.sync_copy(ycbcr.at[pl.ds(in_base + ci * CHUNK, CHUNK)], buf)

        def hist_vec(j, _):
            v = buf[pl.ds(j * 16, 16)]
            t = jnp.clip((v - LO) * INVW, 0.0, NB - 1)
            b = t.astype(jnp.int32)
            plsc.addupdate_scatter(hist_lanes, [lane_off + b], ones)
            return _

        lax.fori_loop(0, CHUNK // 16, hist_vec, None)
        return _

    lax.fori_loop(0, NCHUNK, hist_chunk, None)

    def red_body(m, _):
        acc = zeros
        for l in range(16):
            acc = acc + hist_lanes[pl.ds(l * NB + m * 16, 16)]
        hist_own[pl.ds(m * 16, 16)] = acc
        return _

    lax.fori_loop(0, NB // 16, red_body, None)
    pltpu.sync_copy(hist_own, hists.at[pl.ds(row * NB, NB)])

    plsc.subcore_barrier()
    pltpu.sync_copy(hists.at[pl.ds(prow * NB, NB)], hist_ref)

    def make_cumsum(src, dst):
        def cs_body(m, carry):
            v = src[pl.ds(m * 16, 16)]
            cs = plsc.cumsum(v)
            dst[pl.ds(m * 16, 16)] = cs - v + carry
            return carry + jnp.sum(v)

        total = lax.fori_loop(0, NB // 16, cs_body, jnp.float32(0.0))
        dst[pl.ds(NB, 16)] = zeros + total

    make_cumsum(hist_own, cum_img)
    make_cumsum(hist_ref, cum_ref)

    def lut_body(m, _):
        r = cum_img[pl.ds(m * 16, 16)]
        j = jnp.zeros((16,), jnp.int32)
        st = NB // 2
        while st >= 1:
            cand = j + st
            cv = plsc.load_gather(cum_ref, [cand])
            j = jnp.where((cand <= NB - 1) & (cv <= r), cand, j)
            st //= 2
        c0 = plsc.load_gather(cum_ref, [j])
        c1 = plsc.load_gather(cum_ref, [j + 1])
        cnt = jnp.maximum(c1 - c0, 1.0)
        frac = jnp.clip((r - c0) / cnt, 0.0, 1.0)
        lut[pl.ds(m * 16, 16)] = LO + (j.astype(jnp.float32) + frac) * WBIN
        return _

    lax.fori_loop(0, (NB + 16) // 16, lut_body, None)

    def apply_chunk(ci, _):
        pltpu.sync_copy(ycbcr.at[pl.ds(in_base + ci * CHUNK, CHUNK)], buf)

        def apply_vec(j, _):
            v = buf[pl.ds(j * 16, 16)]
            t = jnp.clip((v - LO) * INVW, 0.0, NB - 1)
            b = t.astype(jnp.int32)
            f = jnp.clip(t - b.astype(jnp.float32), 0.0, 1.0)
            l0 = plsc.load_gather(lut, [b])
            l1 = plsc.load_gather(lut, [b + 1])
            buf[pl.ds(j * 16, 16)] = l0 + f * (l1 - l0)
            return _

        lax.fori_loop(0, CHUNK // 16, apply_vec, None)
        pltpu.sync_copy(buf, cbcr_out.at[pl.ds(out_base + ci * CHUNK, CHUNK)])
        return _

    lax.fori_loop(0, NCHUNK, apply_chunk, None)


_sc_match = functools.partial(
    pl.kernel,
    out_type=(
        jax.ShapeDtypeStruct((NSEL * 2 * HW,), jnp.float32),
        jax.ShapeDtypeStruct((NSEL * 2 * NB,), jnp.float32),
    ),
    mesh=plsc.VectorSubcoreMesh(core_axis_name="c", subcore_axis_name="s"),
    compiler_params=pltpu.CompilerParams(needs_layout_passes=False),
    scratch_types=[
        pltpu.VMEM((CHUNK,), jnp.float32),
        pltpu.VMEM((16 * NB,), jnp.float32),
        pltpu.VMEM((NB,), jnp.float32),
        pltpu.VMEM((NB,), jnp.float32),
        pltpu.VMEM((NB + 16,), jnp.float32),
        pltpu.VMEM((NB + 16,), jnp.float32),
        pltpu.VMEM((NB + 16,), jnp.float32),
    ],
)(_sc_body)


def kernel(x):
    perm = jax.random.permutation(jax.random.key(1), x.shape[0])
    sel = jnp.concatenate([perm[:_K], perm[-_K:]])
    xs = x[sel]
    ycbcr = _rgb2ycbcr(xs)
    cbcr_flat, _ = _sc_match(ycbcr.reshape(-1))
    cbcr = cbcr_flat.reshape(NSEL, 2, H, W)
    rgb_new = _recon(ycbcr, cbcr)
    return x.at[sel].set(rgb_new)


# speedup vs baseline: 56.6676x; 1.3428x over previous
import functools

import jax
import jax.numpy as jnp
import numpy as np
from jax import lax
from jax.experimental import pallas as pl
from jax.experimental.pallas import tpu as pltpu
from jax.experimental.pallas import tpu_sc as plsc

H = W = 512
HW = H * W
NSEL = 16
NB = 2048
LO = -0.25
HI = 1.25
WBIN = (HI - LO) / NB
INVW = 1.0 / WBIN
CHUNK = 8192
NCHUNK = HW // CHUNK
ROWS = 128

_K = 8


def _rgb2ycbcr_body(xb, ob):
    r = xb[0, 0]
    g = xb[0, 1]
    b = xb[0, 2]
    y = 0.299 * r + 0.587 * g + 0.114 * b
    ob[0, 0] = y
    ob[0, 1] = (b - y) * 0.564 + 0.5
    ob[0, 2] = (r - y) * 0.713 + 0.5


_rgb2ycbcr = pl.pallas_call(
    _rgb2ycbcr_body,
    grid=(NSEL, H // ROWS),
    in_specs=[pl.BlockSpec((1, 3, ROWS, W), lambda i, j: (i, 0, j, 0))],
    out_specs=pl.BlockSpec((1, 3, ROWS, W), lambda i, j: (i, 0, j, 0)),
    out_shape=jax.ShapeDtypeStruct((NSEL, 3, H, W), jnp.float32),
)


def _recon_body(yb, cbcrb, ob):
    y = yb[0, 0]
    cb = cbcrb[0, 0] - 0.5
    cr = cbcrb[0, 1] - 0.5
    ob[0, 0] = y + 1.403 * cr
    ob[0, 1] = y - 0.714 * cr - 0.344 * cb
    ob[0, 2] = y + 1.773 * cb


_recon = pl.pallas_call(
    _recon_body,
    grid=(NSEL, H // ROWS),
    in_specs=[
        pl.BlockSpec((1, 1, ROWS, W), lambda i, j: (i, 0, j, 0)),
        pl.BlockSpec((1, 2, ROWS, W), lambda i, j: (i, 0, j, 0)),
    ],
    out_specs=pl.BlockSpec((1, 3, ROWS, W), lambda i, j: (i, 0, j, 0)),
    out_shape=jax.ShapeDtypeStruct((NSEL, 3, H, W), jnp.float32),
)


def _sc_body(ycbcr, cbcr_out, hists, ibuf0, ibuf1, obuf0, obuf1, hist_lanes,
             hist_own, hist_ref, cum_img, cum_ref, lut, semi0, semi1, semo0,
             semo1):
    c = lax.axis_index("c")
    s = lax.axis_index("s")
    k_img = (s // 2) * 2 + c
    chan = s % 2
    k_ref = (k_img + 8) % 16
    row = k_img * 2 + chan
    prow = k_ref * 2 + chan

    lane = lax.iota(jnp.int32, 16)
    lane_off = lane * NB
    ones = jnp.full((16,), 1.0, jnp.float32)
    zeros = jnp.zeros((16,), jnp.float32)
    in_base = (k_img * 3 + 1 + chan) * HW
    out_base = (k_img * 2 + chan) * HW

    def src_at(ci):
        return ycbcr.at[pl.ds(in_base + ci * CHUNK, CHUNK)]

    def dst_at(ci):
        return cbcr_out.at[pl.ds(out_base + ci * CHUNK, CHUNK)]

    def zero_body(i, _):
        for u in range(4):
            hist_lanes[pl.ds(i * 64 + u * 16, 16)] = zeros
        return _

    lax.fori_loop(0, NB // 4, zero_body, None)

    def hist_vecs(bref):
        def hist_vec(j, _):
            for u in range(4):
                v = bref[pl.ds(j * 64 + u * 16, 16)]
                t = jnp.clip((v - LO) * INVW, 0.0, NB - 1)
                b = t.astype(jnp.int32)
                plsc.addupdate_scatter(hist_lanes, [lane_off + b], ones)
            return _

        lax.fori_loop(0, CHUNK // 64, hist_vec, None)

    pltpu.async_copy(src_at(0), ibuf0, semi0)

    def hist_chunk2(i, _):
        ci0 = 2 * i
        ci1 = ci0 + 1
        pltpu.make_async_copy(src_at(ci0), ibuf0, semi0).wait()
        pltpu.async_copy(src_at(ci1), ibuf1, semi1)
        hist_vecs(ibuf0)
        pltpu.make_async_copy(src_at(ci1), ibuf1, semi1).wait()

        @pl.when(ci1 + 1 < NCHUNK)
        def _start_next():
            pltpu.async_copy(src_at(ci1 + 1), ibuf0, semi0)

        hist_vecs(ibuf1)
        return _

    lax.fori_loop(0, NCHUNK // 2, hist_chunk2, None)

    def red_body(m, _):
        acc = zeros
        for l in range(16):
            acc = acc + hist_lanes[pl.ds(l * NB + m * 16, 16)]
        hist_own[pl.ds(m * 16, 16)] = acc
        return _

    lax.fori_loop(0, NB // 16, red_body, None)
    pltpu.sync_copy(hist_own, hists.at[pl.ds(row * NB, NB)])

    plsc.subcore_barrier()
    pltpu.sync_copy(hists.at[pl.ds(prow * NB, NB)], hist_ref)

    def make_cumsum(src, dst):
        def cs_body(m, carry):
            v = src[pl.ds(m * 16, 16)]
            cs = plsc.cumsum(v)
            dst[pl.ds(m * 16, 16)] = cs - v + carry
            return carry + jnp.sum(v)

        total = lax.fori_loop(0, NB // 16, cs_body, jnp.float32(0.0))
        dst[pl.ds(NB, 16)] = zeros + total

    make_cumsum(hist_own, cum_img)
    make_cumsum(hist_ref, cum_ref)

    def lut_body(m, _):
        r = cum_img[pl.ds(m * 16, 16)]
        j = jnp.zeros((16,), jnp.int32)
        st = NB // 2
        while st >= 1:
            cand = j + st
            cv = plsc.load_gather(cum_ref, [cand])
            j = jnp.where((cand <= NB - 1) & (cv <= r), cand, j)
            st //= 2
        c0 = plsc.load_gather(cum_ref, [j])
        c1 = plsc.load_gather(cum_ref, [j + 1])
        cnt = jnp.maximum(c1 - c0, 1.0)
        frac = jnp.clip((r - c0) / cnt, 0.0, 1.0)
        lut[pl.ds(m * 16, 16)] = LO + (j.astype(jnp.float32) + frac) * WBIN
        return _

    lax.fori_loop(0, (NB + 16) // 16, lut_body, None)

    def apply_vecs(bin_ref, bout_ref):
        def apply_vec(j, _):
            for u in range(4):
                v = bin_ref[pl.ds(j * 64 + u * 16, 16)]
                t = jnp.clip((v - LO) * INVW, 0.0, NB - 1)
                b = t.astype(jnp.int32)
                f = jnp.clip(t - b.astype(jnp.float32), 0.0, 1.0)
                l0 = plsc.load_gather(lut, [b])
                l1 = plsc.load_gather(lut, [b + 1])
                bout_ref[pl.ds(j * 64 + u * 16, 16)] = l0 + f * (l1 - l0)
            return _

        lax.fori_loop(0, CHUNK // 64, apply_vec, None)

    pltpu.async_copy(src_at(0), ibuf0, semi0)

    def apply_chunk2(i, _):
        ci0 = 2 * i
        ci1 = ci0 + 1
        pltpu.make_async_copy(src_at(ci0), ibuf0, semi0).wait()
        pltpu.async_copy(src_at(ci1), ibuf1, semi1)

        @pl.when(i > 0)
        def _wait_o0():
            pltpu.make_async_copy(obuf0, dst_at(ci0 - 2), semo0).wait()

        apply_vecs(ibuf0, obuf0)
        pltpu.async_copy(obuf0, dst_at(ci0), semo0)
        pltpu.make_async_copy(src_at(ci1), ibuf1, semi1).wait()

        @pl.when(ci1 + 1 < NCHUNK)
        def _start_next():
            pltpu.async_copy(src_at(ci1 + 1), ibuf0, semi0)

        @pl.when(i > 0)
        def _wait_o1():
            pltpu.make_async_copy(obuf1, dst_at(ci1 - 2), semo1).wait()

        apply_vecs(ibuf1, obuf1)
        pltpu.async_copy(obuf1, dst_at(ci1), semo1)
        return _

    lax.fori_loop(0, NCHUNK // 2, apply_chunk2, None)
    pltpu.make_async_copy(obuf0, dst_at(NCHUNK - 2), semo0).wait()
    pltpu.make_async_copy(obuf1, dst_at(NCHUNK - 1), semo1).wait()


_sc_match = functools.partial(
    pl.kernel,
    out_type=(
        jax.ShapeDtypeStruct((NSEL * 2 * HW,), jnp.float32),
        jax.ShapeDtypeStruct((NSEL * 2 * NB,), jnp.float32),
    ),
    mesh=plsc.VectorSubcoreMesh(core_axis_name="c", subcore_axis_name="s"),
    compiler_params=pltpu.CompilerParams(needs_layout_passes=False),
    scratch_types=[
        pltpu.VMEM((CHUNK,), jnp.float32),
        pltpu.VMEM((CHUNK,), jnp.float32),
        pltpu.VMEM((CHUNK,), jnp.float32),
        pltpu.VMEM((CHUNK,), jnp.float32),
        pltpu.VMEM((16 * NB,), jnp.float32),
        pltpu.VMEM((NB,), jnp.float32),
        pltpu.VMEM((NB,), jnp.float32),
        pltpu.VMEM((NB + 16,), jnp.float32),
        pltpu.VMEM((NB + 16,), jnp.float32),
        pltpu.VMEM((NB + 16,), jnp.float32),
        pltpu.SemaphoreType.DMA,
        pltpu.SemaphoreType.DMA,
        pltpu.SemaphoreType.DMA,
        pltpu.SemaphoreType.DMA,
    ],
)(_sc_body)


def kernel(x):
    perm = jax.random.permutation(jax.random.key(1), x.shape[0])
    sel = jnp.concatenate([perm[:_K], perm[-_K:]])
    xs = x[sel]
    ycbcr = _rgb2ycbcr(xs)
    cbcr_flat, _ = _sc_match(ycbcr.reshape(-1))
    cbcr = cbcr_flat.reshape(NSEL, 2, H, W)
    rgb_new = _recon(ycbcr, cbcr)
    return x.at[sel].set(rgb_new)


# speedup vs baseline: 89.3389x; 1.5765x over previous
import functools

import jax
import jax.numpy as jnp
import numpy as np
from jax import lax
from jax.experimental import pallas as pl
from jax.experimental.pallas import tpu as pltpu
from jax.experimental.pallas import tpu_sc as plsc

H = W = 512
HW = H * W
NSEL = 16
NB = 2048
LO = -0.25
HI = 1.25
WBIN = (HI - LO) / NB
INVW = 1.0 / WBIN
CHUNK = 8192
NCHUNK = HW // CHUNK
ROWS = 128

_K = 8


def _rgb2ycbcr_body(xb, ob):
    r = xb[0, 0]
    g = xb[0, 1]
    b = xb[0, 2]
    y = 0.299 * r + 0.587 * g + 0.114 * b
    ob[0, 0] = y
    ob[0, 1] = (b - y) * 0.564 + 0.5
    ob[0, 2] = (r - y) * 0.713 + 0.5


_rgb2ycbcr = pl.pallas_call(
    _rgb2ycbcr_body,
    grid=(NSEL, H // ROWS),
    in_specs=[pl.BlockSpec((1, 3, ROWS, W), lambda i, j: (i, 0, j, 0))],
    out_specs=pl.BlockSpec((1, 3, ROWS, W), lambda i, j: (i, 0, j, 0)),
    out_shape=jax.ShapeDtypeStruct((NSEL, 3, H, W), jnp.float32),
)


def _recon_body(yb, cbcrb, ob):
    y = yb[0, 0]
    cb = cbcrb[0, 0] - 0.5
    cr = cbcrb[0, 1] - 0.5
    ob[0, 0] = y + 1.403 * cr
    ob[0, 1] = y - 0.714 * cr - 0.344 * cb
    ob[0, 2] = y + 1.773 * cb


_recon = pl.pallas_call(
    _recon_body,
    grid=(NSEL, H // ROWS),
    in_specs=[
        pl.BlockSpec((1, 1, ROWS, W), lambda i, j: (i, 0, j, 0)),
        pl.BlockSpec((1, 2, ROWS, W), lambda i, j: (i, 0, j, 0)),
    ],
    out_specs=pl.BlockSpec((1, 3, ROWS, W), lambda i, j: (i, 0, j, 0)),
    out_shape=jax.ShapeDtypeStruct((NSEL, 3, H, W), jnp.float32),
)


def _sc_body(ycbcr, cbcr_out, hists, ibuf0, ibuf1, obuf0, obuf1, hist_lanes,
             hist_own, hist_ref, cum_img, cum_ref, lut, semi0, semi1, semo0,
             semo1):
    c = lax.axis_index("c")
    s = lax.axis_index("s")
    k_img = (s // 2) * 2 + c
    chan = s % 2
    k_ref = (k_img + 8) % 16
    row = k_img * 2 + chan
    prow = k_ref * 2 + chan

    lane = lax.iota(jnp.int32, 16)
    lane_off = lane * NB
    ones = jnp.full((16,), 1.0, jnp.float32)
    zeros = jnp.zeros((16,), jnp.float32)
    in_base = (k_img * 3 + 1 + chan) * HW
    out_base = (k_img * 2 + chan) * HW

    def src_at(ci):
        return ycbcr.at[pl.ds(in_base + ci * CHUNK, CHUNK)]

    def dst_at(ci):
        return cbcr_out.at[pl.ds(out_base + ci * CHUNK, CHUNK)]

    @plsc.parallel_loop(0, NB, unroll=8)
    def _zero_body(i):
        hist_lanes[pl.ds(i * 16, 16)] = zeros

    def hist_vecs(bref):
        @plsc.parallel_loop(0, CHUNK // 16, unroll=8)
        def _hist_vec(j):
            v = bref[pl.ds(j * 16, 16)]
            t = jnp.clip((v - LO) * INVW, 0.0, NB - 1)
            b = t.astype(jnp.int32)
            plsc.addupdate_scatter(hist_lanes, [lane_off + b], ones)

    pltpu.async_copy(src_at(0), ibuf0, semi0)

    def hist_chunk2(i, _):
        ci0 = 2 * i
        ci1 = ci0 + 1
        pltpu.make_async_copy(src_at(ci0), ibuf0, semi0).wait()
        pltpu.async_copy(src_at(ci1), ibuf1, semi1)
        hist_vecs(ibuf0)
        pltpu.make_async_copy(src_at(ci1), ibuf1, semi1).wait()

        @pl.when(ci1 + 1 < NCHUNK)
        def _start_next():
            pltpu.async_copy(src_at(ci1 + 1), ibuf0, semi0)

        hist_vecs(ibuf1)
        return _

    lax.fori_loop(0, NCHUNK // 2, hist_chunk2, None)

    @plsc.parallel_loop(0, NB // 16, unroll=4)
    def _red_body(m):
        acc = zeros
        for l in range(16):
            acc = acc + hist_lanes[pl.ds(l * NB + m * 16, 16)]
        hist_own[pl.ds(m * 16, 16)] = acc
    pltpu.sync_copy(hist_own, hists.at[pl.ds(row * NB, NB)])

    plsc.subcore_barrier()
    pltpu.sync_copy(hists.at[pl.ds(prow * NB, NB)], hist_ref)

    def make_cumsum(src, dst):
        def cs_body(m, carry):
            v = src[pl.ds(m * 16, 16)]
            cs = plsc.cumsum(v)
            dst[pl.ds(m * 16, 16)] = cs - v + carry
            return carry + jnp.sum(v)

        total = lax.fori_loop(0, NB // 16, cs_body, jnp.float32(0.0))
        dst[pl.ds(NB, 16)] = zeros + total

    make_cumsum(hist_own, cum_img)
    make_cumsum(hist_ref, cum_ref)

    def lut_body(m, _):
        r = cum_img[pl.ds(m * 16, 16)]
        j = jnp.zeros((16,), jnp.int32)
        st = NB // 2
        while st >= 1:
            cand = j + st
            cv = plsc.load_gather(cum_ref, [cand])
            j = jnp.where((cand <= NB - 1) & (cv <= r), cand, j)
            st //= 2
        c0 = plsc.load_gather(cum_ref, [j])
        c1 = plsc.load_gather(cum_ref, [j + 1])
        cnt = jnp.maximum(c1 - c0, 1.0)
        frac = jnp.clip((r - c0) / cnt, 0.0, 1.0)
        lut[pl.ds(m * 16, 16)] = LO + (j.astype(jnp.float32) + frac) * WBIN
        return _

    lax.fori_loop(0, (NB + 16) // 16, lut_body, None)

    def apply_vecs(bin_ref, bout_ref):
        @plsc.parallel_loop(0, CHUNK // 16, unroll=8)
        def _apply_vec(j):
            v = bin_ref[pl.ds(j * 16, 16)]
            t = jnp.clip((v - LO) * INVW, 0.0, NB - 1)
            b = t.astype(jnp.int32)
            f = jnp.clip(t - b.astype(jnp.float32), 0.0, 1.0)
            l0 = plsc.load_gather(lut, [b])
            l1 = plsc.load_gather(lut, [b + 1])
            bout_ref[pl.ds(j * 16, 16)] = l0 + f * (l1 - l0)

    pltpu.async_copy(src_at(0), ibuf0, semi0)

    def apply_chunk2(i, _):
        ci0 = 2 * i
        ci1 = ci0 + 1
        pltpu.make_async_copy(src_at(ci0), ibuf0, semi0).wait()
        pltpu.async_copy(src_at(ci1), ibuf1, semi1)

        @pl.when(i > 0)
        def _wait_o0():
            pltpu.make_async_copy(obuf0, dst_at(ci0 - 2), semo0).wait()

        apply_vecs(ibuf0, obuf0)
        pltpu.async_copy(obuf0, dst_at(ci0), semo0)
        pltpu.make_async_copy(src_at(ci1), ibuf1, semi1).wait()

        @pl.when(ci1 + 1 < NCHUNK)
        def _start_next():
            pltpu.async_copy(src_at(ci1 + 1), ibuf0, semi0)

        @pl.when(i > 0)
        def _wait_o1():
            pltpu.make_async_copy(obuf1, dst_at(ci1 - 2), semo1).wait()

        apply_vecs(ibuf1, obuf1)
        pltpu.async_copy(obuf1, dst_at(ci1), semo1)
        return _

    lax.fori_loop(0, NCHUNK // 2, apply_chunk2, None)
    pltpu.make_async_copy(obuf0, dst_at(NCHUNK - 2), semo0).wait()
    pltpu.make_async_copy(obuf1, dst_at(NCHUNK - 1), semo1).wait()


_sc_match = functools.partial(
    pl.kernel,
    out_type=(
        jax.ShapeDtypeStruct((NSEL * 2 * HW,), jnp.float32),
        jax.ShapeDtypeStruct((NSEL * 2 * NB,), jnp.float32),
    ),
    mesh=plsc.VectorSubcoreMesh(core_axis_name="c", subcore_axis_name="s"),
    compiler_params=pltpu.CompilerParams(needs_layout_passes=False),
    scratch_types=[
        pltpu.VMEM((CHUNK,), jnp.float32),
        pltpu.VMEM((CHUNK,), jnp.float32),
        pltpu.VMEM((CHUNK,), jnp.float32),
        pltpu.VMEM((CHUNK,), jnp.float32),
        pltpu.VMEM((16 * NB,), jnp.float32),
        pltpu.VMEM((NB,), jnp.float32),
        pltpu.VMEM((NB,), jnp.float32),
        pltpu.VMEM((NB + 16,), jnp.float32),
        pltpu.VMEM((NB + 16,), jnp.float32),
        pltpu.VMEM((NB + 16,), jnp.float32),
        pltpu.SemaphoreType.DMA,
        pltpu.SemaphoreType.DMA,
        pltpu.SemaphoreType.DMA,
        pltpu.SemaphoreType.DMA,
    ],
)(_sc_body)


def kernel(x):
    perm = jax.random.permutation(jax.random.key(1), x.shape[0])
    sel = jnp.concatenate([perm[:_K], perm[-_K:]])
    xs = x[sel]
    ycbcr = _rgb2ycbcr(xs)
    cbcr_flat, _ = _sc_match(ycbcr.reshape(-1))
    cbcr = cbcr_flat.reshape(NSEL, 2, H, W)
    rgb_new = _recon(ycbcr, cbcr)
    return x.at[sel].set(rgb_new)


# speedup vs baseline: 90.3432x; 1.0112x over previous
import functools

import jax
import jax.numpy as jnp
import numpy as np
from jax import lax
from jax.experimental import pallas as pl
from jax.experimental.pallas import tpu as pltpu
from jax.experimental.pallas import tpu_sc as plsc

H = W = 512
HW = H * W
NSEL = 16
NB = 2048
LO = -0.25
HI = 1.25
WBIN = (HI - LO) / NB
INVW = 1.0 / WBIN
CHUNK = 8192
NCHUNK = HW // CHUNK
ROWS = 128

_K = 8


def _rgb2ycbcr_body(xb, ob):
    r = xb[0, 0]
    g = xb[0, 1]
    b = xb[0, 2]
    y = 0.299 * r + 0.587 * g + 0.114 * b
    ob[0, 0] = y
    ob[0, 1] = (b - y) * 0.564 + 0.5
    ob[0, 2] = (r - y) * 0.713 + 0.5


_rgb2ycbcr = pl.pallas_call(
    _rgb2ycbcr_body,
    grid=(NSEL, H // ROWS),
    in_specs=[pl.BlockSpec((1, 3, ROWS, W), lambda i, j: (i, 0, j, 0))],
    out_specs=pl.BlockSpec((1, 3, ROWS, W), lambda i, j: (i, 0, j, 0)),
    out_shape=jax.ShapeDtypeStruct((NSEL, 3, H, W), jnp.float32),
)


def _recon_body(yb, cbcrb, ob):
    y = yb[0, 0]
    cb = cbcrb[0, 0] - 0.5
    cr = cbcrb[0, 1] - 0.5
    ob[0, 0] = y + 1.403 * cr
    ob[0, 1] = y - 0.714 * cr - 0.344 * cb
    ob[0, 2] = y + 1.773 * cb


_recon = pl.pallas_call(
    _recon_body,
    grid=(NSEL, H // ROWS),
    in_specs=[
        pl.BlockSpec((1, 1, ROWS, W), lambda i, j: (i, 0, j, 0)),
        pl.BlockSpec((1, 2, ROWS, W), lambda i, j: (i, 0, j, 0)),
    ],
    out_specs=pl.BlockSpec((1, 3, ROWS, W), lambda i, j: (i, 0, j, 0)),
    out_shape=jax.ShapeDtypeStruct((NSEL, 3, H, W), jnp.float32),
)


def _sc_body(ycbcr, cbcr_out, hists, ibuf0, ibuf1, obuf0, obuf1, hist_lanes,
             hist_own, hist_ref, cum_img, cum_ref, lut, semi0, semi1, semo0,
             semo1):
    c = lax.axis_index("c")
    s = lax.axis_index("s")
    k_img = (s // 2) * 2 + c
    chan = s % 2
    k_ref = (k_img + 8) % 16
    row = k_img * 2 + chan
    prow = k_ref * 2 + chan

    lane = lax.iota(jnp.int32, 16)
    lane_off = lane * NB
    ones = jnp.full((16,), 1.0, jnp.float32)
    zeros = jnp.zeros((16,), jnp.float32)
    in_base = (k_img * 3 + 1 + chan) * HW
    out_base = (k_img * 2 + chan) * HW

    def src_at(ci):
        return ycbcr.at[pl.ds(in_base + ci * CHUNK, CHUNK)]

    def dst_at(ci):
        return cbcr_out.at[pl.ds(out_base + ci * CHUNK, CHUNK)]

    @plsc.parallel_loop(0, NB, unroll=8)
    def _zero_body(i):
        hist_lanes[pl.ds(i * 16, 16)] = zeros

    def hist_vecs(bref):
        @plsc.parallel_loop(0, CHUNK // 16, unroll=8)
        def _hist_vec(j):
            v = bref[pl.ds(j * 16, 16)]
            t = jnp.clip((v - LO) * INVW, 0.0, NB - 1)
            b = t.astype(jnp.int32)
            plsc.addupdate_scatter(hist_lanes, [lane_off + b], ones)

    pltpu.async_copy(src_at(0), ibuf0, semi0)

    def hist_chunk2(i, _):
        ci0 = 2 * i
        ci1 = ci0 + 1
        pltpu.make_async_copy(src_at(ci0), ibuf0, semi0).wait()
        pltpu.async_copy(src_at(ci1), ibuf1, semi1)
        hist_vecs(ibuf0)
        pltpu.make_async_copy(src_at(ci1), ibuf1, semi1).wait()

        @pl.when(ci1 + 1 < NCHUNK)
        def _start_next():
            pltpu.async_copy(src_at(ci1 + 1), ibuf0, semi0)

        hist_vecs(ibuf1)
        return _

    lax.fori_loop(0, NCHUNK // 2, hist_chunk2, None)

    @plsc.parallel_loop(0, NB // 16, unroll=4)
    def _red_body(m):
        acc = zeros
        for l in range(16):
            acc = acc + hist_lanes[pl.ds(l * NB + m * 16, 16)]
        hist_own[pl.ds(m * 16, 16)] = acc
    pltpu.sync_copy(hist_own, hists.at[pl.ds(row * NB, NB)])

    plsc.subcore_barrier()
    pltpu.sync_copy(hists.at[pl.ds(prow * NB, NB)], hist_ref)

    def make_cumsum(src, dst):
        def cs_body(m, carry):
            v = src[pl.ds(m * 16, 16)]
            cs = plsc.cumsum(v)
            dst[pl.ds(m * 16, 16)] = cs - v + carry
            return carry + jnp.sum(v)

        total = lax.fori_loop(0, NB // 16, cs_body, jnp.float32(0.0))
        dst[pl.ds(NB, 16)] = zeros + total

    make_cumsum(hist_own, cum_img)
    make_cumsum(hist_ref, cum_ref)

    def lut_body(m, _):
        r = cum_img[pl.ds(m * 16, 16)]
        j = jnp.zeros((16,), jnp.int32)
        st = NB // 2
        while st >= 1:
            cand = j + st
            cv = plsc.load_gather(cum_ref, [cand])
            j = jnp.where((cand <= NB - 1) & (cv <= r), cand, j)
            st //= 2
        c0 = plsc.load_gather(cum_ref, [j])
        c1 = plsc.load_gather(cum_ref, [j + 1])
        cnt = jnp.maximum(c1 - c0, 1.0)
        frac = jnp.clip((r - c0) / cnt, 0.0, 1.0)
        lut[pl.ds(m * 16, 16)] = LO + (j.astype(jnp.float32) + frac) * WBIN
        return _

    lax.fori_loop(0, (NB + 16) // 16, lut_body, None)

    def apply_vecs(bin_ref, bout_ref):
        @plsc.parallel_loop(0, CHUNK // 16, unroll=8)
        def _apply_vec(j):
            v = bin_ref[pl.ds(j * 16, 16)]
            t = jnp.clip((v - LO) * INVW, 0.0, NB - 1)
            b = t.astype(jnp.int32)
            f = jnp.clip(t - b.astype(jnp.float32), 0.0, 1.0)
            l0 = plsc.load_gather(lut, [b])
            l1 = plsc.load_gather(lut, [b + 1])
            bout_ref[pl.ds(j * 16, 16)] = l0 + f * (l1 - l0)

    pltpu.async_copy(src_at(0), ibuf0, semi0)

    def apply_chunk2(i, _):
        ci0 = 2 * i
        ci1 = ci0 + 1
        pltpu.make_async_copy(src_at(ci0), ibuf0, semi0).wait()
        pltpu.async_copy(src_at(ci1), ibuf1, semi1)

        @pl.when(i > 0)
        def _wait_o0():
            pltpu.make_async_copy(obuf0, dst_at(ci0 - 2), semo0).wait()

        apply_vecs(ibuf0, obuf0)
        pltpu.async_copy(obuf0, dst_at(ci0), semo0)
        pltpu.make_async_copy(src_at(ci1), ibuf1, semi1).wait()

        @pl.when(ci1 + 1 < NCHUNK)
        def _start_next():
            pltpu.async_copy(src_at(ci1 + 1), ibuf0, semi0)

        @pl.when(i > 0)
        def _wait_o1():
            pltpu.make_async_copy(obuf1, dst_at(ci1 - 2), semo1).wait()

        apply_vecs(ibuf1, obuf1)
        pltpu.async_copy(obuf1, dst_at(ci1), semo1)
        return _

    lax.fori_loop(0, NCHUNK // 2, apply_chunk2, None)
    pltpu.make_async_copy(obuf0, dst_at(NCHUNK - 2), semo0).wait()
    pltpu.make_async_copy(obuf1, dst_at(NCHUNK - 1), semo1).wait()


_sc_match = functools.partial(
    pl.kernel,
    out_type=(
        jax.ShapeDtypeStruct((NSEL * 2 * HW,), jnp.float32),
        jax.ShapeDtypeStruct((NSEL * 2 * NB,), jnp.float32),
    ),
    mesh=plsc.VectorSubcoreMesh(core_axis_name="c", subcore_axis_name="s"),
    compiler_params=pltpu.CompilerParams(needs_layout_passes=False),
    scratch_types=[
        pltpu.VMEM((CHUNK,), jnp.float32),
        pltpu.VMEM((CHUNK,), jnp.float32),
        pltpu.VMEM((CHUNK,), jnp.float32),
        pltpu.VMEM((CHUNK,), jnp.float32),
        pltpu.VMEM((16 * NB,), jnp.float32),
        pltpu.VMEM((NB,), jnp.float32),
        pltpu.VMEM((NB,), jnp.float32),
        pltpu.VMEM((NB + 16,), jnp.float32),
        pltpu.VMEM((NB + 16,), jnp.float32),
        pltpu.VMEM((NB + 16,), jnp.float32),
        pltpu.SemaphoreType.DMA,
        pltpu.SemaphoreType.DMA,
        pltpu.SemaphoreType.DMA,
        pltpu.SemaphoreType.DMA,
    ],
)(_sc_body)


def kernel(x):
    with jax.ensure_compile_time_eval():
        perm = np.asarray(jax.random.permutation(jax.random.key(1), x.shape[0]))
    sel = np.concatenate([perm[:_K], perm[-_K:]])
    ycbcr = _rgb2ycbcr(x[sel])
    cbcr_flat, _ = _sc_match(ycbcr.reshape(-1))
    cbcr = cbcr_flat.reshape(NSEL, 2, H, W)
    return x.at[sel].set(_recon(ycbcr, cbcr))


# speedup vs baseline: 126.9264x; 1.4049x over previous
import functools

import jax
import jax.numpy as jnp
import numpy as np
from jax import lax
from jax.experimental import pallas as pl
from jax.experimental.pallas import tpu as pltpu
from jax.experimental.pallas import tpu_sc as plsc

H = W = 512
HW = H * W
NSEL = 16
NB = 2048
LO = -0.25
HI = 1.25
WBIN = (HI - LO) / NB
INVW = 1.0 / WBIN
CHUNK = 8192
NCHUNK = HW // CHUNK
ROWS = 256

_K = 8


def _rgb2ycbcr_body(sel_ref, xb, ob):
    del sel_ref
    r = xb[0, 0]
    g = xb[0, 1]
    b = xb[0, 2]
    y = 0.299 * r + 0.587 * g + 0.114 * b
    ob[0, 0] = y
    ob[0, 1] = (b - y) * 0.564 + 0.5
    ob[0, 2] = (r - y) * 0.713 + 0.5


_rgb2ycbcr = pl.pallas_call(
    _rgb2ycbcr_body,
    grid_spec=pltpu.PrefetchScalarGridSpec(
        num_scalar_prefetch=1,
        grid=(NSEL, H // ROWS),
        in_specs=[
            pl.BlockSpec((1, 3, ROWS, W), lambda i, j, sel: (sel[i], 0, j, 0))
        ],
        out_specs=pl.BlockSpec((1, 3, ROWS, W), lambda i, j, sel: (i, 0, j, 0)),
    ),
    out_shape=jax.ShapeDtypeStruct((NSEL, 3, H, W), jnp.float32),
)


def _recon_body(sel_ref, xb, yb, cbcrb, ob):
    del sel_ref, xb
    y = yb[0, 0]
    cb = cbcrb[0, 0] - 0.5
    cr = cbcrb[0, 1] - 0.5
    ob[0, 0] = y + 1.403 * cr
    ob[0, 1] = y - 0.714 * cr - 0.344 * cb
    ob[0, 2] = y + 1.773 * cb


_recon = pl.pallas_call(
    _recon_body,
    grid_spec=pltpu.PrefetchScalarGridSpec(
        num_scalar_prefetch=1,
        grid=(NSEL, H // ROWS),
        in_specs=[
            pl.BlockSpec(memory_space=pl.ANY),
            pl.BlockSpec((1, 1, ROWS, W), lambda i, j, sel: (i, 0, j, 0)),
            pl.BlockSpec((1, 2, ROWS, W), lambda i, j, sel: (i, 0, j, 0)),
        ],
        out_specs=pl.BlockSpec(
            (1, 3, ROWS, W), lambda i, j, sel: (sel[i], 0, j, 0)
        ),
    ),
    out_shape=jax.ShapeDtypeStruct((32, 3, H, W), jnp.float32),
    input_output_aliases={1: 0},
)


def _sc_body(ycbcr, cbcr_out, hists, ibuf0, ibuf1, obuf0, obuf1, hist_lanes,
             hist_own, hist_ref, cum_img, cum_ref, lut, semi0, semi1, semo0,
             semo1):
    c = lax.axis_index("c")
    s = lax.axis_index("s")
    k_img = (s // 2) * 2 + c
    chan = s % 2
    k_ref = (k_img + 8) % 16
    row = k_img * 2 + chan
    prow = k_ref * 2 + chan

    lane = lax.iota(jnp.int32, 16)
    lane_off = lane * NB
    ones = jnp.full((16,), 1.0, jnp.float32)
    zeros = jnp.zeros((16,), jnp.float32)
    in_base = (k_img * 3 + 1 + chan) * HW
    out_base = (k_img * 2 + chan) * HW

    def src_at(ci):
        return ycbcr.at[pl.ds(in_base + ci * CHUNK, CHUNK)]

    def dst_at(ci):
        return cbcr_out.at[pl.ds(out_base + ci * CHUNK, CHUNK)]

    @plsc.parallel_loop(0, NB, unroll=8)
    def _zero_body(i):
        hist_lanes[pl.ds(i * 16, 16)] = zeros

    def hist_vecs(bref):
        @plsc.parallel_loop(0, CHUNK // 16, unroll=8)
        def _hist_vec(j):
            v = bref[pl.ds(j * 16, 16)]
            t = jnp.clip((v - LO) * INVW, 0.0, NB - 1)
            b = t.astype(jnp.int32)
            plsc.addupdate_scatter(hist_lanes, [lane_off + b], ones)

    pltpu.async_copy(src_at(0), ibuf0, semi0)

    def hist_chunk2(i, _):
        ci0 = 2 * i
        ci1 = ci0 + 1
        pltpu.make_async_copy(src_at(ci0), ibuf0, semi0).wait()
        pltpu.async_copy(src_at(ci1), ibuf1, semi1)
        hist_vecs(ibuf0)
        pltpu.make_async_copy(src_at(ci1), ibuf1, semi1).wait()

        @pl.when(ci1 + 1 < NCHUNK)
        def _start_next():
            pltpu.async_copy(src_at(ci1 + 1), ibuf0, semi0)

        hist_vecs(ibuf1)
        return _

    lax.fori_loop(0, NCHUNK // 2, hist_chunk2, None)

    @plsc.parallel_loop(0, NB // 16, unroll=4)
    def _red_body(m):
        acc = zeros
        for l in range(16):
            acc = acc + hist_lanes[pl.ds(l * NB + m * 16, 16)]
        hist_own[pl.ds(m * 16, 16)] = acc
    pltpu.sync_copy(hist_own, hists.at[pl.ds(row * NB, NB)])

    plsc.subcore_barrier()
    pltpu.sync_copy(hists.at[pl.ds(prow * NB, NB)], hist_ref)

    def make_cumsum(src, dst):
        def cs_body(m, carry):
            v = src[pl.ds(m * 16, 16)]
            cs = plsc.cumsum(v)
            dst[pl.ds(m * 16, 16)] = cs - v + carry
            return carry + jnp.sum(v)

        total = lax.fori_loop(0, NB // 16, cs_body, jnp.float32(0.0))
        dst[pl.ds(NB, 16)] = zeros + total

    make_cumsum(hist_own, cum_img)
    make_cumsum(hist_ref, cum_ref)

    def lut_body(m, _):
        r = cum_img[pl.ds(m * 16, 16)]
        j = jnp.zeros((16,), jnp.int32)
        st = NB // 2
        while st >= 1:
            cand = j + st
            cv = plsc.load_gather(cum_ref, [cand])
            j = jnp.where((cand <= NB - 1) & (cv <= r), cand, j)
            st //= 2
        c0 = plsc.load_gather(cum_ref, [j])
        c1 = plsc.load_gather(cum_ref, [j + 1])
        cnt = jnp.maximum(c1 - c0, 1.0)
        frac = jnp.clip((r - c0) / cnt, 0.0, 1.0)
        lut[pl.ds(m * 16, 16)] = LO + (j.astype(jnp.float32) + frac) * WBIN
        return _

    lax.fori_loop(0, (NB + 16) // 16, lut_body, None)

    def apply_vecs(bin_ref, bout_ref):
        @plsc.parallel_loop(0, CHUNK // 16, unroll=8)
        def _apply_vec(j):
            v = bin_ref[pl.ds(j * 16, 16)]
            t = jnp.clip((v - LO) * INVW, 0.0, NB - 1)
            b = t.astype(jnp.int32)
            f = jnp.clip(t - b.astype(jnp.float32), 0.0, 1.0)
            l0 = plsc.load_gather(lut, [b])
            l1 = plsc.load_gather(lut, [b + 1])
            bout_ref[pl.ds(j * 16, 16)] = l0 + f * (l1 - l0)

    pltpu.async_copy(src_at(0), ibuf0, semi0)

    def apply_chunk2(i, _):
        ci0 = 2 * i
        ci1 = ci0 + 1
        pltpu.make_async_copy(src_at(ci0), ibuf0, semi0).wait()
        pltpu.async_copy(src_at(ci1), ibuf1, semi1)

        @pl.when(i > 0)
        def _wait_o0():
            pltpu.make_async_copy(obuf0, dst_at(ci0 - 2), semo0).wait()

        apply_vecs(ibuf0, obuf0)
        pltpu.async_copy(obuf0, dst_at(ci0), semo0)
        pltpu.make_async_copy(src_at(ci1), ibuf1, semi1).wait()

        @pl.when(ci1 + 1 < NCHUNK)
        def _start_next():
            pltpu.async_copy(src_at(ci1 + 1), ibuf0, semi0)

        @pl.when(i > 0)
        def _wait_o1():
            pltpu.make_async_copy(obuf1, dst_at(ci1 - 2), semo1).wait()

        apply_vecs(ibuf1, obuf1)
        pltpu.async_copy(obuf1, dst_at(ci1), semo1)
        return _

    lax.fori_loop(0, NCHUNK // 2, apply_chunk2, None)
    pltpu.make_async_copy(obuf0, dst_at(NCHUNK - 2), semo0).wait()
    pltpu.make_async_copy(obuf1, dst_at(NCHUNK - 1), semo1).wait()


_sc_match = functools.partial(
    pl.kernel,
    out_type=(
        jax.ShapeDtypeStruct((NSEL * 2 * HW,), jnp.float32),
        jax.ShapeDtypeStruct((NSEL * 2 * NB,), jnp.float32),
    ),
    mesh=plsc.VectorSubcoreMesh(core_axis_name="c", subcore_axis_name="s"),
    compiler_params=pltpu.CompilerParams(needs_layout_passes=False),
    scratch_types=[
        pltpu.VMEM((CHUNK,), jnp.float32),
        pltpu.VMEM((CHUNK,), jnp.float32),
        pltpu.VMEM((CHUNK,), jnp.float32),
        pltpu.VMEM((CHUNK,), jnp.float32),
        pltpu.VMEM((16 * NB,), jnp.float32),
        pltpu.VMEM((NB,), jnp.float32),
        pltpu.VMEM((NB,), jnp.float32),
        pltpu.VMEM((NB + 16,), jnp.float32),
        pltpu.VMEM((NB + 16,), jnp.float32),
        pltpu.VMEM((NB + 16,), jnp.float32),
        pltpu.SemaphoreType.DMA,
        pltpu.SemaphoreType.DMA,
        pltpu.SemaphoreType.DMA,
        pltpu.SemaphoreType.DMA,
    ],
)(_sc_body)


def kernel(x):
    with jax.ensure_compile_time_eval():
        perm = np.asarray(jax.random.permutation(jax.random.key(1), x.shape[0]))
    sel = jnp.asarray(np.concatenate([perm[:_K], perm[-_K:]]), jnp.int32)
    ycbcr = _rgb2ycbcr(sel, x)
    cbcr_flat, _ = _sc_match(ycbcr.reshape(-1))
    cbcr = cbcr_flat.reshape(NSEL, 2, H, W)
    return _recon(sel, x, ycbcr, cbcr)


# speedup vs baseline: 131.3489x; 1.0348x over previous
import functools

import jax
import jax.numpy as jnp
import numpy as np
from jax import lax
from jax.experimental import pallas as pl
from jax.experimental.pallas import tpu as pltpu
from jax.experimental.pallas import tpu_sc as plsc

H = W = 512
HW = H * W
NSEL = 16
NB = 2048
LO = -0.25
HI = 1.25
WBIN = (HI - LO) / NB
INVW = 1.0 / WBIN
CHUNK = 8192
NCHUNK = HW // CHUNK
ROWS = 512

_K = 8


def _rgb2ycbcr_body(sel_ref, xb, ob):
    del sel_ref
    r = xb[0, 0]
    g = xb[0, 1]
    b = xb[0, 2]
    y = 0.299 * r + 0.587 * g + 0.114 * b
    ob[0, 0] = y
    ob[0, 1] = (b - y) * 0.564 + 0.5
    ob[0, 2] = (r - y) * 0.713 + 0.5


_rgb2ycbcr = pl.pallas_call(
    _rgb2ycbcr_body,
    grid_spec=pltpu.PrefetchScalarGridSpec(
        num_scalar_prefetch=1,
        grid=(NSEL, H // ROWS),
        in_specs=[
            pl.BlockSpec((1, 3, ROWS, W), lambda i, j, sel: (sel[i], 0, j, 0))
        ],
        out_specs=pl.BlockSpec((1, 3, ROWS, W), lambda i, j, sel: (i, 0, j, 0)),
    ),
    out_shape=jax.ShapeDtypeStruct((NSEL, 3, H, W), jnp.float32),
)


def _recon_body(sel_ref, xb, yb, cbcrb, ob):
    del sel_ref, xb
    y = yb[0, 0]
    cb = cbcrb[0, 0] - 0.5
    cr = cbcrb[0, 1] - 0.5
    ob[0, 0] = y + 1.403 * cr
    ob[0, 1] = y - 0.714 * cr - 0.344 * cb
    ob[0, 2] = y + 1.773 * cb


_recon = pl.pallas_call(
    _recon_body,
    grid_spec=pltpu.PrefetchScalarGridSpec(
        num_scalar_prefetch=1,
        grid=(NSEL, H // ROWS),
        in_specs=[
            pl.BlockSpec(memory_space=pl.ANY),
            pl.BlockSpec((1, 1, ROWS, W), lambda i, j, sel: (i, 0, j, 0)),
            pl.BlockSpec((1, 2, ROWS, W), lambda i, j, sel: (i, 0, j, 0)),
        ],
        out_specs=pl.BlockSpec(
            (1, 3, ROWS, W), lambda i, j, sel: (sel[i], 0, j, 0)
        ),
    ),
    out_shape=jax.ShapeDtypeStruct((32, 3, H, W), jnp.float32),
    input_output_aliases={1: 0},
)


def _sc_body(ycbcr, cbcr_out, hists, ibuf0, ibuf1, obuf0, obuf1, hist_lanes,
             hist_own, hist_ref, cum_img, cum_ref, lut, semi0, semi1, semo0,
             semo1):
    c = lax.axis_index("c")
    s = lax.axis_index("s")
    k_img = (s // 2) * 2 + c
    chan = s % 2
    k_ref = (k_img + 8) % 16
    row = k_img * 2 + chan
    prow = k_ref * 2 + chan

    lane = lax.iota(jnp.int32, 16)
    lane_off = lane * NB
    ones = jnp.full((16,), 1.0, jnp.float32)
    zeros = jnp.zeros((16,), jnp.float32)
    in_base = (k_img * 3 + 1 + chan) * HW
    out_base = (k_img * 2 + chan) * HW

    def src_at(ci):
        return ycbcr.at[pl.ds(in_base + ci * CHUNK, CHUNK)]

    def dst_at(ci):
        return cbcr_out.at[pl.ds(out_base + ci * CHUNK, CHUNK)]

    @plsc.parallel_loop(0, NB, unroll=8)
    def _zero_body(i):
        hist_lanes[pl.ds(i * 16, 16)] = zeros

    def hist_vecs(bref):
        @plsc.parallel_loop(0, CHUNK // 16, unroll=16)
        def _hist_vec(j):
            v = bref[pl.ds(j * 16, 16)]
            t = jnp.clip((v - LO) * INVW, 0.0, NB - 1)
            b = t.astype(jnp.int32)
            plsc.addupdate_scatter(hist_lanes, [lane_off + b], ones)

    pltpu.async_copy(src_at(0), ibuf0, semi0)

    def hist_chunk2(i, _):
        ci0 = 2 * i
        ci1 = ci0 + 1
        pltpu.make_async_copy(src_at(ci0), ibuf0, semi0).wait()
        pltpu.async_copy(src_at(ci1), ibuf1, semi1)
        hist_vecs(ibuf0)
        pltpu.make_async_copy(src_at(ci1), ibuf1, semi1).wait()

        @pl.when(ci1 + 1 < NCHUNK)
        def _start_next():
            pltpu.async_copy(src_at(ci1 + 1), ibuf0, semi0)

        hist_vecs(ibuf1)
        return _

    lax.fori_loop(0, NCHUNK // 2, hist_chunk2, None)

    @plsc.parallel_loop(0, NB // 16, unroll=4)
    def _red_body(m):
        acc = zeros
        for l in range(16):
            acc = acc + hist_lanes[pl.ds(l * NB + m * 16, 16)]
        hist_own[pl.ds(m * 16, 16)] = acc
    pltpu.sync_copy(hist_own, hists.at[pl.ds(row * NB, NB)])

    plsc.subcore_barrier()
    pltpu.sync_copy(hists.at[pl.ds(prow * NB, NB)], hist_ref)

    def make_cumsum(src, dst):
        def cs_body(m, carry):
            v = src[pl.ds(m * 16, 16)]
            cs = plsc.cumsum(v)
            dst[pl.ds(m * 16, 16)] = cs - v + carry
            return carry + jnp.sum(v)

        total = lax.fori_loop(0, NB // 16, cs_body, jnp.float32(0.0))
        dst[pl.ds(NB, 16)] = zeros + total

    make_cumsum(hist_own, cum_img)
    make_cumsum(hist_ref, cum_ref)

    def lut_body(m, _):
        r = cum_img[pl.ds(m * 16, 16)]
        j = jnp.zeros((16,), jnp.int32)
        st = NB // 2
        while st >= 1:
            cand = j + st
            cv = plsc.load_gather(cum_ref, [cand])
            j = jnp.where((cand <= NB - 1) & (cv <= r), cand, j)
            st //= 2
        c0 = plsc.load_gather(cum_ref, [j])
        c1 = plsc.load_gather(cum_ref, [j + 1])
        cnt = jnp.maximum(c1 - c0, 1.0)
        frac = jnp.clip((r - c0) / cnt, 0.0, 1.0)
        lut[pl.ds(m * 16, 16)] = LO + (j.astype(jnp.float32) + frac) * WBIN
        return _

    lax.fori_loop(0, (NB + 16) // 16, lut_body, None)

    def apply_vecs(bin_ref, bout_ref):
        @plsc.parallel_loop(0, CHUNK // 16, unroll=16)
        def _apply_vec(j):
            v = bin_ref[pl.ds(j * 16, 16)]
            t = jnp.clip((v - LO) * INVW, 0.0, NB - 1)
            b = t.astype(jnp.int32)
            f = jnp.clip(t - b.astype(jnp.float32), 0.0, 1.0)
            l0 = plsc.load_gather(lut, [b])
            l1 = plsc.load_gather(lut, [b + 1])
            bout_ref[pl.ds(j * 16, 16)] = l0 + f * (l1 - l0)

    pltpu.async_copy(src_at(0), ibuf0, semi0)

    def apply_chunk2(i, _):
        ci0 = 2 * i
        ci1 = ci0 + 1
        pltpu.make_async_copy(src_at(ci0), ibuf0, semi0).wait()
        pltpu.async_copy(src_at(ci1), ibuf1, semi1)

        @pl.when(i > 0)
        def _wait_o0():
            pltpu.make_async_copy(obuf0, dst_at(ci0 - 2), semo0).wait()

        apply_vecs(ibuf0, obuf0)
        pltpu.async_copy(obuf0, dst_at(ci0), semo0)
        pltpu.make_async_copy(src_at(ci1), ibuf1, semi1).wait()

        @pl.when(ci1 + 1 < NCHUNK)
        def _start_next():
            pltpu.async_copy(src_at(ci1 + 1), ibuf0, semi0)

        @pl.when(i > 0)
        def _wait_o1():
            pltpu.make_async_copy(obuf1, dst_at(ci1 - 2), semo1).wait()

        apply_vecs(ibuf1, obuf1)
        pltpu.async_copy(obuf1, dst_at(ci1), semo1)
        return _

    lax.fori_loop(0, NCHUNK // 2, apply_chunk2, None)
    pltpu.make_async_copy(obuf0, dst_at(NCHUNK - 2), semo0).wait()
    pltpu.make_async_copy(obuf1, dst_at(NCHUNK - 1), semo1).wait()


_sc_match = functools.partial(
    pl.kernel,
    out_type=(
        jax.ShapeDtypeStruct((NSEL * 2 * HW,), jnp.float32),
        jax.ShapeDtypeStruct((NSEL * 2 * NB,), jnp.float32),
    ),
    mesh=plsc.VectorSubcoreMesh(core_axis_name="c", subcore_axis_name="s"),
    compiler_params=pltpu.CompilerParams(needs_layout_passes=False),
    scratch_types=[
        pltpu.VMEM((CHUNK,), jnp.float32),
        pltpu.VMEM((CHUNK,), jnp.float32),
        pltpu.VMEM((CHUNK,), jnp.float32),
        pltpu.VMEM((CHUNK,), jnp.float32),
        pltpu.VMEM((16 * NB,), jnp.float32),
        pltpu.VMEM((NB,), jnp.float32),
        pltpu.VMEM((NB,), jnp.float32),
        pltpu.VMEM((NB + 16,), jnp.float32),
        pltpu.VMEM((NB + 16,), jnp.float32),
        pltpu.VMEM((NB + 16,), jnp.float32),
        pltpu.SemaphoreType.DMA,
        pltpu.SemaphoreType.DMA,
        pltpu.SemaphoreType.DMA,
        pltpu.SemaphoreType.DMA,
    ],
)(_sc_body)


def kernel(x):
    with jax.ensure_compile_time_eval():
        perm = np.asarray(jax.random.permutation(jax.random.key(1), x.shape[0]))
    sel = jnp.asarray(np.concatenate([perm[:_K], perm[-_K:]]), jnp.int32)
    ycbcr = _rgb2ycbcr(sel, x)
    cbcr_flat, _ = _sc_match(ycbcr.reshape(-1))
    cbcr = cbcr_flat.reshape(NSEL, 2, H, W)
    return _recon(sel, x, ycbcr, cbcr)


# speedup vs baseline: 142.6277x; 1.0859x over previous
import functools

import jax
import jax.numpy as jnp
import numpy as np
from jax import lax
from jax.experimental import pallas as pl
from jax.experimental.pallas import tpu as pltpu
from jax.experimental.pallas import tpu_sc as plsc

H = W = 512
HW = H * W
NSEL = 16
NB = 2048
LO = -0.25
HI = 1.25
WBIN = (HI - LO) / NB
INVW = 1.0 / WBIN
CHUNK = 8192
NCHUNK = HW // CHUNK
ROWS = 512

_K = 8


def _rgb2ycbcr_body(sel_ref, xb, ob):
    del sel_ref
    r = xb[0, 0]
    g = xb[0, 1]
    b = xb[0, 2]
    y = 0.299 * r + 0.587 * g + 0.114 * b
    ob[0, 0] = y
    ob[0, 1] = (b - y) * 0.564 + 0.5
    ob[0, 2] = (r - y) * 0.713 + 0.5


_rgb2ycbcr = pl.pallas_call(
    _rgb2ycbcr_body,
    grid_spec=pltpu.PrefetchScalarGridSpec(
        num_scalar_prefetch=1,
        grid=(NSEL, H // ROWS),
        in_specs=[
            pl.BlockSpec((1, 3, ROWS, W), lambda i, j, sel: (sel[i], 0, j, 0))
        ],
        out_specs=pl.BlockSpec((1, 3, ROWS, W), lambda i, j, sel: (i, 0, j, 0)),
    ),
    out_shape=jax.ShapeDtypeStruct((NSEL, 3, H, W), jnp.float32),
)


def _copy_body(xb, ob):
    ob[...] = xb[...]


_copy32 = pl.pallas_call(
    _copy_body,
    grid=(32,),
    in_specs=[pl.BlockSpec((1, 3, H, W), lambda i: (i, 0, 0, 0))],
    out_specs=pl.BlockSpec((1, 3, H, W), lambda i: (i, 0, 0, 0)),
    out_shape=jax.ShapeDtypeStruct((32, 3, H, W), jnp.float32),
)


def _recon_body(sel_ref, xb, yb, cbcrb, ob):
    del sel_ref, xb
    y = yb[0, 0]
    cb = cbcrb[0, 0] - 0.5
    cr = cbcrb[0, 1] - 0.5
    ob[0, 0] = y + 1.403 * cr
    ob[0, 1] = y - 0.714 * cr - 0.344 * cb
    ob[0, 2] = y + 1.773 * cb


_recon = pl.pallas_call(
    _recon_body,
    grid_spec=pltpu.PrefetchScalarGridSpec(
        num_scalar_prefetch=1,
        grid=(NSEL, H // ROWS),
        in_specs=[
            pl.BlockSpec(memory_space=pl.ANY),
            pl.BlockSpec((1, 1, ROWS, W), lambda i, j, sel: (i, 0, j, 0)),
            pl.BlockSpec((1, 2, ROWS, W), lambda i, j, sel: (i, 0, j, 0)),
        ],
        out_specs=pl.BlockSpec(
            (1, 3, ROWS, W), lambda i, j, sel: (sel[i], 0, j, 0)
        ),
    ),
    out_shape=jax.ShapeDtypeStruct((32, 3, H, W), jnp.float32),
    input_output_aliases={1: 0},
)


def _sc_body(ycbcr, cbcr_out, hists, ibuf0, ibuf1, obuf0, obuf1, hist_lanes,
             hist_own, hist_ref, cum_img, cum_ref, lut, semi0, semi1, semo0,
             semo1):
    c = lax.axis_index("c")
    s = lax.axis_index("s")
    k_img = (s // 2) * 2 + c
    chan = s % 2
    k_ref = (k_img + 8) % 16
    row = k_img * 2 + chan
    prow = k_ref * 2 + chan

    lane = lax.iota(jnp.int32, 16)
    lane_off = lane * NB
    ones = jnp.full((16,), 1.0, jnp.float32)
    zeros = jnp.zeros((16,), jnp.float32)
    in_base = (k_img * 3 + 1 + chan) * HW
    out_base = (k_img * 2 + chan) * HW

    def src_at(ci):
        return ycbcr.at[pl.ds(in_base + ci * CHUNK, CHUNK)]

    def dst_at(ci):
        return cbcr_out.at[pl.ds(out_base + ci * CHUNK, CHUNK)]

    @plsc.parallel_loop(0, NB, unroll=8)
    def _zero_body(i):
        hist_lanes[pl.ds(i * 16, 16)] = zeros

    def hist_vecs(bref):
        @plsc.parallel_loop(0, CHUNK // 16, unroll=16)
        def _hist_vec(j):
            v = bref[pl.ds(j * 16, 16)]
            t = jnp.clip((v - LO) * INVW, 0.0, NB - 1)
            b = t.astype(jnp.int32)
            plsc.addupdate_scatter(hist_lanes, [lane_off + b], ones)

    pltpu.async_copy(src_at(0), ibuf0, semi0)

    def hist_chunk2(i, _):
        ci0 = 2 * i
        ci1 = ci0 + 1
        pltpu.make_async_copy(src_at(ci0), ibuf0, semi0).wait()
        pltpu.async_copy(src_at(ci1), ibuf1, semi1)
        hist_vecs(ibuf0)
        pltpu.make_async_copy(src_at(ci1), ibuf1, semi1).wait()

        @pl.when(ci1 + 1 < NCHUNK)
        def _start_next():
            pltpu.async_copy(src_at(ci1 + 1), ibuf0, semi0)

        hist_vecs(ibuf1)
        return _

    lax.fori_loop(0, NCHUNK // 2, hist_chunk2, None)

    @plsc.parallel_loop(0, NB // 16, unroll=4)
    def _red_body(m):
        acc = zeros
        for l in range(16):
            acc = acc + hist_lanes[pl.ds(l * NB + m * 16, 16)]
        hist_own[pl.ds(m * 16, 16)] = acc
    pltpu.sync_copy(hist_own, hists.at[pl.ds(row * NB, NB)])

    plsc.subcore_barrier()
    pltpu.sync_copy(hists.at[pl.ds(prow * NB, NB)], hist_ref)

    def make_cumsum(src, dst):
        def cs_body(m, carry):
            v = src[pl.ds(m * 16, 16)]
            cs = plsc.cumsum(v)
            dst[pl.ds(m * 16, 16)] = cs - v + carry
            return carry + jnp.sum(v)

        total = lax.fori_loop(0, NB // 16, cs_body, jnp.float32(0.0))
        dst[pl.ds(NB, 16)] = zeros + total

    make_cumsum(hist_own, cum_img)
    make_cumsum(hist_ref, cum_ref)

    def lut_body(m, _):
        r = cum_img[pl.ds(m * 16, 16)]
        j = jnp.zeros((16,), jnp.int32)
        st = NB // 2
        while st >= 1:
            cand = j + st
            cv = plsc.load_gather(cum_ref, [cand])
            j = jnp.where((cand <= NB - 1) & (cv <= r), cand, j)
            st //= 2
        c0 = plsc.load_gather(cum_ref, [j])
        c1 = plsc.load_gather(cum_ref, [j + 1])
        cnt = jnp.maximum(c1 - c0, 1.0)
        frac = jnp.clip((r - c0) / cnt, 0.0, 1.0)
        lut[pl.ds(m * 16, 16)] = LO + (j.astype(jnp.float32) + frac) * WBIN
        return _

    lax.fori_loop(0, (NB + 16) // 16, lut_body, None)

    def apply_vecs(bin_ref, bout_ref):
        @plsc.parallel_loop(0, CHUNK // 16, unroll=16)
        def _apply_vec(j):
            v = bin_ref[pl.ds(j * 16, 16)]
            t = jnp.clip((v - LO) * INVW, 0.0, NB - 1)
            b = t.astype(jnp.int32)
            f = jnp.clip(t - b.astype(jnp.float32), 0.0, 1.0)
            l0 = plsc.load_gather(lut, [b])
            l1 = plsc.load_gather(lut, [b + 1])
            bout_ref[pl.ds(j * 16, 16)] = l0 + f * (l1 - l0)

    pltpu.async_copy(src_at(0), ibuf0, semi0)

    def apply_chunk2(i, _):
        ci0 = 2 * i
        ci1 = ci0 + 1
        pltpu.make_async_copy(src_at(ci0), ibuf0, semi0).wait()
        pltpu.async_copy(src_at(ci1), ibuf1, semi1)

        @pl.when(i > 0)
        def _wait_o0():
            pltpu.make_async_copy(obuf0, dst_at(ci0 - 2), semo0).wait()

        apply_vecs(ibuf0, obuf0)
        pltpu.async_copy(obuf0, dst_at(ci0), semo0)
        pltpu.make_async_copy(src_at(ci1), ibuf1, semi1).wait()

        @pl.when(ci1 + 1 < NCHUNK)
        def _start_next():
            pltpu.async_copy(src_at(ci1 + 1), ibuf0, semi0)

        @pl.when(i > 0)
        def _wait_o1():
            pltpu.make_async_copy(obuf1, dst_at(ci1 - 2), semo1).wait()

        apply_vecs(ibuf1, obuf1)
        pltpu.async_copy(obuf1, dst_at(ci1), semo1)
        return _

    lax.fori_loop(0, NCHUNK // 2, apply_chunk2, None)
    pltpu.make_async_copy(obuf0, dst_at(NCHUNK - 2), semo0).wait()
    pltpu.make_async_copy(obuf1, dst_at(NCHUNK - 1), semo1).wait()


_sc_match = functools.partial(
    pl.kernel,
    out_type=(
        jax.ShapeDtypeStruct((NSEL * 2 * HW,), jnp.float32),
        jax.ShapeDtypeStruct((NSEL * 2 * NB,), jnp.float32),
    ),
    mesh=plsc.VectorSubcoreMesh(core_axis_name="c", subcore_axis_name="s"),
    compiler_params=pltpu.CompilerParams(needs_layout_passes=False),
    scratch_types=[
        pltpu.VMEM((CHUNK,), jnp.float32),
        pltpu.VMEM((CHUNK,), jnp.float32),
        pltpu.VMEM((CHUNK,), jnp.float32),
        pltpu.VMEM((CHUNK,), jnp.float32),
        pltpu.VMEM((16 * NB,), jnp.float32),
        pltpu.VMEM((NB,), jnp.float32),
        pltpu.VMEM((NB,), jnp.float32),
        pltpu.VMEM((NB + 16,), jnp.float32),
        pltpu.VMEM((NB + 16,), jnp.float32),
        pltpu.VMEM((NB + 16,), jnp.float32),
        pltpu.SemaphoreType.DMA,
        pltpu.SemaphoreType.DMA,
        pltpu.SemaphoreType.DMA,
        pltpu.SemaphoreType.DMA,
    ],
)(_sc_body)


def kernel(x):
    with jax.ensure_compile_time_eval():
        perm = np.asarray(jax.random.permutation(jax.random.key(1), x.shape[0]))
    sel = jnp.asarray(np.concatenate([perm[:_K], perm[-_K:]]), jnp.int32)
    out_init = _copy32(x)
    ycbcr = _rgb2ycbcr(sel, x)
    cbcr_flat, _ = _sc_match(ycbcr.reshape(-1))
    cbcr = cbcr_flat.reshape(NSEL, 2, H, W)
    return _recon(sel, out_init, ycbcr, cbcr)


# speedup vs baseline: 152.9568x; 1.0724x over previous
import functools

import jax
import jax.numpy as jnp
import numpy as np
from jax import lax
from jax.experimental import pallas as pl
from jax.experimental.pallas import tpu as pltpu
from jax.experimental.pallas import tpu_sc as plsc

H = W = 512
HW = H * W
NSEL = 16
NB = 2048
LO = -0.25
HI = 1.25
WBIN = (HI - LO) / NB
INVW = 1.0 / WBIN
CHUNK = 16384
NCHUNK = HW // CHUNK
ROWS = 512

_K = 8


def _rgb2ycbcr_body(sel_ref, xb, ob):
    del sel_ref
    r = xb[0, 0]
    g = xb[0, 1]
    b = xb[0, 2]
    y = 0.299 * r + 0.587 * g + 0.114 * b
    ob[0, 0] = y
    ob[0, 1] = (b - y) * 0.564 + 0.5
    ob[0, 2] = (r - y) * 0.713 + 0.5


_rgb2ycbcr = pl.pallas_call(
    _rgb2ycbcr_body,
    grid_spec=pltpu.PrefetchScalarGridSpec(
        num_scalar_prefetch=1,
        grid=(NSEL, H // ROWS),
        in_specs=[
            pl.BlockSpec((1, 3, ROWS, W), lambda i, j, sel: (sel[i], 0, j, 0))
        ],
        out_specs=pl.BlockSpec((1, 3, ROWS, W), lambda i, j, sel: (i, 0, j, 0)),
    ),
    out_shape=jax.ShapeDtypeStruct((NSEL, 3, H, W), jnp.float32),
)


def _copy_body(xb, ob):
    ob[...] = xb[...]


_copy32 = pl.pallas_call(
    _copy_body,
    grid=(32,),
    in_specs=[pl.BlockSpec((1, 3, H, W), lambda i: (i, 0, 0, 0))],
    out_specs=pl.BlockSpec((1, 3, H, W), lambda i: (i, 0, 0, 0)),
    out_shape=jax.ShapeDtypeStruct((32, 3, H, W), jnp.float32),
)


def _recon_body(sel_ref, xb, yb, cbcrb, ob):
    del sel_ref, xb
    y = yb[0, 0]
    cb = cbcrb[0, 0] - 0.5
    cr = cbcrb[0, 1] - 0.5
    ob[0, 0] = y + 1.403 * cr
    ob[0, 1] = y - 0.714 * cr - 0.344 * cb
    ob[0, 2] = y + 1.773 * cb


_recon = pl.pallas_call(
    _recon_body,
    grid_spec=pltpu.PrefetchScalarGridSpec(
        num_scalar_prefetch=1,
        grid=(NSEL, H // ROWS),
        in_specs=[
            pl.BlockSpec(memory_space=pl.ANY),
            pl.BlockSpec((1, 1, ROWS, W), lambda i, j, sel: (i, 0, j, 0)),
            pl.BlockSpec((1, 2, ROWS, W), lambda i, j, sel: (i, 0, j, 0)),
        ],
        out_specs=pl.BlockSpec(
            (1, 3, ROWS, W), lambda i, j, sel: (sel[i], 0, j, 0)
        ),
    ),
    out_shape=jax.ShapeDtypeStruct((32, 3, H, W), jnp.float32),
    input_output_aliases={1: 0},
)


def _sc_body(ycbcr, cbcr_out, hists, ibuf0, ibuf1, obuf0, obuf1, hist_lanes,
             hist_own, hist_ref, cum_img, cum_ref, lut, semi0, semi1, semo0,
             semo1):
    c = lax.axis_index("c")
    s = lax.axis_index("s")
    k_img = (s // 2) * 2 + c
    chan = s % 2
    k_ref = (k_img + 8) % 16
    row = k_img * 2 + chan
    prow = k_ref * 2 + chan

    lane = lax.iota(jnp.int32, 16)
    lane_off = lane * NB
    ones = jnp.full((16,), 1.0, jnp.float32)
    zeros = jnp.zeros((16,), jnp.float32)
    in_base = (k_img * 3 + 1 + chan) * HW
    out_base = (k_img * 2 + chan) * HW

    def src_at(ci):
        return ycbcr.at[pl.ds(in_base + ci * CHUNK, CHUNK)]

    def dst_at(ci):
        return cbcr_out.at[pl.ds(out_base + ci * CHUNK, CHUNK)]

    @plsc.parallel_loop(0, NB, unroll=8)
    def _zero_body(i):
        hist_lanes[pl.ds(i * 16, 16)] = zeros

    def hist_vecs(bref):
        @plsc.parallel_loop(0, CHUNK // 16, unroll=16)
        def _hist_vec(j):
            v = bref[pl.ds(j * 16, 16)]
            t = jnp.clip((v - LO) * INVW, 0.0, NB - 1)
            b = t.astype(jnp.int32)
            plsc.addupdate_scatter(hist_lanes, [lane_off + b], ones)

    pltpu.async_copy(src_at(0), ibuf0, semi0)

    def hist_chunk2(i, _):
        ci0 = 2 * i
        ci1 = ci0 + 1
        pltpu.make_async_copy(src_at(ci0), ibuf0, semi0).wait()
        pltpu.async_copy(src_at(ci1), ibuf1, semi1)
        hist_vecs(ibuf0)
        pltpu.make_async_copy(src_at(ci1), ibuf1, semi1).wait()

        @pl.when(ci1 + 1 < NCHUNK)
        def _start_next():
            pltpu.async_copy(src_at(ci1 + 1), ibuf0, semi0)

        hist_vecs(ibuf1)
        return _

    lax.fori_loop(0, NCHUNK // 2, hist_chunk2, None)

    @plsc.parallel_loop(0, NB // 16, unroll=4)
    def _red_body(m):
        acc = zeros
        for l in range(16):
            acc = acc + hist_lanes[pl.ds(l * NB + m * 16, 16)]
        hist_own[pl.ds(m * 16, 16)] = acc
    pltpu.sync_copy(hist_own, hists.at[pl.ds(row * NB, NB)])

    plsc.subcore_barrier()
    pltpu.sync_copy(hists.at[pl.ds(prow * NB, NB)], hist_ref)

    def make_cumsum(src, dst):
        def cs_body(m, carry):
            v = src[pl.ds(m * 16, 16)]
            cs = plsc.cumsum(v)
            dst[pl.ds(m * 16, 16)] = cs - v + carry
            return carry + jnp.sum(v)

        total = lax.fori_loop(0, NB // 16, cs_body, jnp.float32(0.0))
        dst[pl.ds(NB, 16)] = zeros + total

    make_cumsum(hist_own, cum_img)
    make_cumsum(hist_ref, cum_ref)

    def lut_body(m, _):
        r = cum_img[pl.ds(m * 16, 16)]
        j = jnp.zeros((16,), jnp.int32)
        st = NB // 2
        while st >= 1:
            cand = j + st
            cv = plsc.load_gather(cum_ref, [cand])
            j = jnp.where((cand <= NB - 1) & (cv <= r), cand, j)
            st //= 2
        c0 = plsc.load_gather(cum_ref, [j])
        c1 = plsc.load_gather(cum_ref, [j + 1])
        cnt = jnp.maximum(c1 - c0, 1.0)
        frac = jnp.clip((r - c0) / cnt, 0.0, 1.0)
        lut[pl.ds(m * 16, 16)] = LO + (j.astype(jnp.float32) + frac) * WBIN
        return _

    lax.fori_loop(0, (NB + 16) // 16, lut_body, None)

    def apply_vecs(bin_ref, bout_ref):
        @plsc.parallel_loop(0, CHUNK // 16, unroll=16)
        def _apply_vec(j):
            v = bin_ref[pl.ds(j * 16, 16)]
            t = jnp.clip((v - LO) * INVW, 0.0, NB - 1)
            b = t.astype(jnp.int32)
            f = jnp.clip(t - b.astype(jnp.float32), 0.0, 1.0)
            l0 = plsc.load_gather(lut, [b])
            l1 = plsc.load_gather(lut, [b + 1])
            bout_ref[pl.ds(j * 16, 16)] = l0 + f * (l1 - l0)

    pltpu.async_copy(src_at(0), ibuf0, semi0)

    def apply_chunk2(i, _):
        ci0 = 2 * i
        ci1 = ci0 + 1
        pltpu.make_async_copy(src_at(ci0), ibuf0, semi0).wait()
        pltpu.async_copy(src_at(ci1), ibuf1, semi1)

        @pl.when(i > 0)
        def _wait_o0():
            pltpu.make_async_copy(obuf0, dst_at(ci0 - 2), semo0).wait()

        apply_vecs(ibuf0, obuf0)
        pltpu.async_copy(obuf0, dst_at(ci0), semo0)
        pltpu.make_async_copy(src_at(ci1), ibuf1, semi1).wait()

        @pl.when(ci1 + 1 < NCHUNK)
        def _start_next():
            pltpu.async_copy(src_at(ci1 + 1), ibuf0, semi0)

        @pl.when(i > 0)
        def _wait_o1():
            pltpu.make_async_copy(obuf1, dst_at(ci1 - 2), semo1).wait()

        apply_vecs(ibuf1, obuf1)
        pltpu.async_copy(obuf1, dst_at(ci1), semo1)
        return _

    lax.fori_loop(0, NCHUNK // 2, apply_chunk2, None)
    pltpu.make_async_copy(obuf0, dst_at(NCHUNK - 2), semo0).wait()
    pltpu.make_async_copy(obuf1, dst_at(NCHUNK - 1), semo1).wait()


_sc_match = functools.partial(
    pl.kernel,
    out_type=(
        jax.ShapeDtypeStruct((NSEL * 2 * HW,), jnp.float32),
        jax.ShapeDtypeStruct((NSEL * 2 * NB,), jnp.float32),
    ),
    mesh=plsc.VectorSubcoreMesh(core_axis_name="c", subcore_axis_name="s"),
    compiler_params=pltpu.CompilerParams(needs_layout_passes=False),
    scratch_types=[
        pltpu.VMEM((CHUNK,), jnp.float32),
        pltpu.VMEM((CHUNK,), jnp.float32),
        pltpu.VMEM((CHUNK,), jnp.float32),
        pltpu.VMEM((CHUNK,), jnp.float32),
        pltpu.VMEM((16 * NB,), jnp.float32),
        pltpu.VMEM((NB,), jnp.float32),
        pltpu.VMEM((NB,), jnp.float32),
        pltpu.VMEM((NB + 16,), jnp.float32),
        pltpu.VMEM((NB + 16,), jnp.float32),
        pltpu.VMEM((NB + 16,), jnp.float32),
        pltpu.SemaphoreType.DMA,
        pltpu.SemaphoreType.DMA,
        pltpu.SemaphoreType.DMA,
        pltpu.SemaphoreType.DMA,
    ],
)(_sc_body)


def kernel(x):
    with jax.ensure_compile_time_eval():
        perm = np.asarray(jax.random.permutation(jax.random.key(1), x.shape[0]))
    sel = jnp.asarray(np.concatenate([perm[:_K], perm[-_K:]]), jnp.int32)
    out_init = _copy32(x)
    ycbcr = _rgb2ycbcr(sel, x)
    cbcr_flat, _ = _sc_match(ycbcr.reshape(-1))
    cbcr = cbcr_flat.reshape(NSEL, 2, H, W)
    return _recon(sel, out_init, ycbcr, cbcr)
